# baseline (speedup 1.0000x reference)
import functools

import jax
import jax.numpy as jnp
from jax.experimental import pallas as pl
from jax.experimental.pallas import tpu as pltpu
from jax.experimental.pallas import tpu_sc as plsc


def _stage_a_body(f2m_ref, maskf_ref, c2_ref, f_ref, cb_ref,
                  assign_ref, rowmin_ref, colmin_ref, rm_s, ra_s,
                  *, bn, bk, kb_total):
    nb = pl.program_id(0)
    kb = pl.program_id(1)
    ns = bk // 128

    fm2 = -2.0 * f_ref[...]

    f2m = f2m_ref[0]
    c2 = c2_ref[0, 0, :]

    @pl.when(kb == 0)
    def _():
        rm_s[...] = jnp.full((bn, 128), 3e38, jnp.float32)
        ra_s[...] = jnp.zeros((bn, 128), jnp.int32)

    m = rm_s[...]
    a = ra_s[...]
    cols = []
    half = bk // 2
    nhs = half // 128
    for h in range(2):
        cbh = cb_ref[pl.ds(h * half, half), :]
        t = jax.lax.dot_general(
            fm2, cbh, (((1,), (1,)), ((), ())),
            preferred_element_type=jnp.float32)
        for sh in range(nhs):
            s = h * nhs + sh
            ts = t[:, sh * 128:(sh + 1) * 128]
            ds = (f2m + ts) + c2[s * 128:(s + 1) * 128][None, :]
            better = ds < m
            m = jnp.where(better, ds, m)
            a = jnp.where(better, jnp.int32(kb * ns + s), a)
            cols.append(jnp.min(ds, axis=0)[None, :])
    rm_s[...] = m
    ra_s[...] = a

    tcol = jnp.concatenate(cols, axis=1)

    @pl.when(nb == 0)
    def _():
        colmin_ref[pl.ds(kb, 1), :] = tcol

    @pl.when(nb > 0)
    def _():
        old = colmin_ref[pl.ds(kb, 1), :]
        colmin_ref[pl.ds(kb, 1), :] = jnp.minimum(old, tcol)

    @pl.when(kb == kb_total - 1)
    def _():
        mfin = rm_s[...]
        afin = ra_s[...]
        tmin = jnp.min(mfin, axis=1)
        lane = jax.lax.broadcasted_iota(jnp.int32, (bn, 128), 1)
        kfull = afin * 128 + lane
        targ = jnp.min(jnp.where(mfin == tmin[:, None], kfull,
                                 jnp.int32(2**30)), axis=1)
        mk = maskf_ref[0, 0, :]
        valid = mk > 0.0
        assign_ref[0, 0, :] = jnp.where(valid, targ, 0)
        rowmin_ref[0, 0, :] = jnp.where(valid, tmin, 0.0)


def _stage_a(features, maskf, cb, f2m, c2, bn, bk):
    n, d_dim = features.shape
    k = cb.shape[0]
    nb_total, kb_total = n // bn, k // bk
    body = functools.partial(_stage_a_body, bn=bn, bk=bk, kb_total=kb_total)
    assign3, rowmin3, colmin2 = pl.pallas_call(
        body,
        grid=(nb_total, kb_total),
        in_specs=[
            pl.BlockSpec((1, bn, 1), lambda nb, kb: (nb, 0, 0)),
            pl.BlockSpec((1, 1, bn), lambda nb, kb: (nb, 0, 0)),
            pl.BlockSpec((1, 1, bk), lambda nb, kb: (kb, 0, 0)),
            pl.BlockSpec((bn, d_dim), lambda nb, kb: (nb, 0)),
            pl.BlockSpec((bk, d_dim), lambda nb, kb: (kb, 0)),
        ],
        out_specs=[
            pl.BlockSpec((1, 1, bn), lambda nb, kb: (nb, 0, 0)),
            pl.BlockSpec((1, 1, bn), lambda nb, kb: (nb, 0, 0)),
            pl.BlockSpec((kb_total, bk), lambda nb, kb: (0, 0)),
        ],
        out_shape=[
            jax.ShapeDtypeStruct((nb_total, 1, bn), jnp.int32),
            jax.ShapeDtypeStruct((nb_total, 1, bn), jnp.float32),
            jax.ShapeDtypeStruct((kb_total, bk), jnp.float32),
        ],
        scratch_shapes=[
            pltpu.VMEM((bn, 128), jnp.float32),
            pltpu.VMEM((bn, 128), jnp.int32),
        ],
    )(f2m.reshape(nb_total, bn, 1), maskf.reshape(nb_total, 1, bn),
      c2.reshape(kb_total, 1, bk), features, cb)
    return (assign3.reshape(n), rowmin3.reshape(n), colmin2.reshape(k))


_SC_CHUNK = 128


def _make_stage_b(n, d_dim, k):
    info = plsc.get_sparse_core_info()
    nc, ns = info.num_cores, info.num_subcores
    rows_per_w = n // (nc * ns)
    chunks = rows_per_w // _SC_CHUNK
    mesh = plsc.VectorSubcoreMesh(core_axis_name="c", subcore_axis_name="s")

    @functools.partial(
        pl.kernel, mesh=mesh,
        out_type=[
            jax.ShapeDtypeStruct((n, d_dim), jnp.float32),
            jax.ShapeDtypeStruct((nc, k), jnp.float32),
        ],
        scratch_types=[
            pltpu.VMEM((rows_per_w,), jnp.int32),
            pltpu.VMEM((rows_per_w,), jnp.float32),
            pltpu.VMEM((2, _SC_CHUNK, d_dim), jnp.float32),
            pltpu.VMEM_SHARED((k,), jnp.float32),
            pltpu.SemaphoreType.DMA,
            pltpu.SemaphoreType.DMA,
            pltpu.SemaphoreType.DMA,
            pltpu.SemaphoreType.DMA,
            pltpu.SemaphoreType.DMA,
    ],
    )
    def stage_b(cb_hbm, assign_hbm, maskf_hbm, zeros_hbm,
                outf_hbm, cnt_hbm, idx_v, val_v, rows_v, shist,
                gs0, gs1, ws0, ws1, hs):
        cid = jax.lax.axis_index("c")
        sid = jax.lax.axis_index("s")
        base = (sid * nc + cid) * rows_per_w
        gsems = (gs0, gs1)
        wsems = (ws0, ws1)

        @pl.when(sid == 0)
        def _():
            pltpu.sync_copy(zeros_hbm, shist)

        pltpu.sync_copy(assign_hbm.at[pl.ds(base, rows_per_w)], idx_v)
        pltpu.sync_copy(maskf_hbm.at[pl.ds(base, rows_per_w)], val_v)
        plsc.subcore_barrier()

        hadd = pltpu.async_copy(val_v, shist.at[idx_v], hs, add=True)

        gets = [None, None]
        puts = [None, None]
        for c in range(chunks):
            b = c % 2
            if puts[b] is not None:
                puts[b].wait()
            gets[b] = pltpu.async_copy(
                cb_hbm.at[idx_v.at[pl.ds(c * _SC_CHUNK, _SC_CHUNK)]],
                rows_v.at[b], gsems[b])
            gets[b].wait()
            puts[b] = pltpu.async_copy(
                rows_v.at[b],
                outf_hbm.at[pl.ds(base + c * _SC_CHUNK, _SC_CHUNK)],
                wsems[b])
        for b in range(2):
            if puts[b] is not None:
                puts[b].wait()
        hadd.wait()

        plsc.subcore_barrier()

        @pl.when(sid == 0)
        def _():
            pltpu.sync_copy(shist, cnt_hbm.at[cid])

    return stage_b


def _stage_c_body(rowmin_ref, maskf_ref, colmin_ref, cnt_ref,
                  cb_loss_ref, cm_loss_ref, ul_ref, pct_ref, *, d_dim, k):
    rm_sum = jnp.sum(rowmin_ref[...])
    nvalid = jnp.sum(maskf_ref[...])
    loss = rm_sum / jnp.float32(d_dim) / jnp.maximum(nvalid, 1.0)
    cb_loss_ref[...] = loss.reshape(1, 1)
    cm_loss_ref[...] = loss.reshape(1, 1)

    cnt = jnp.sum(cnt_ref[...], axis=0)
    colmin = colmin_ref[...].reshape(cnt.shape)
    um = jnp.where(cnt < 1.0, 1.0, 0.0)
    denom = jnp.maximum(jnp.sum(um), 1.0)
    ul = jnp.sum(um * colmin) / jnp.float32(d_dim) / denom
    ul_ref[...] = ul.reshape(1, 1)
    pct = jnp.sum(jnp.where(cnt > 0.0, 1.0, 0.0)) / jnp.float32(k)
    pct_ref[...] = pct.reshape(1, 1)


def _stage_c(rowmin, maskf, colmin, cnts, d_dim, k):
    n = rowmin.shape[0]
    body = functools.partial(_stage_c_body, d_dim=d_dim, k=k)
    outs = pl.pallas_call(
        body,
        out_shape=[jax.ShapeDtypeStruct((1, 1), jnp.float32)] * 4,
    )(rowmin.reshape(n // 128, 128), maskf.reshape(n // 128, 128),
      colmin.reshape(k // 128, 128), cnts)
    return [o.reshape(()) for o in outs]


def kernel(features, mask, codebook, codebook_mean, codebook_scale):
    n, d_dim = features.shape
    k = codebook.shape[0]

    cb = 10.0 * codebook
    scale = jnp.exp(codebook_scale)
    cb = codebook_mean + scale * cb
    f2 = (features ** 2).sum(axis=-1)
    c2 = (cb ** 2).sum(axis=-1)
    maskf = mask.astype(jnp.float32)
    f2m = jnp.where(mask, f2, jnp.float32(1e30))

    assign, rowmin, colmin = _stage_a(features, maskf, cb, f2m, c2,
                                      bn=1024, bk=2048)

    zeros_k = jnp.zeros((k,), jnp.float32)
    out_features, cnts = _make_stage_b(n, d_dim, k)(cb, assign, maskf, zeros_k)

    cb_loss, cm_loss, ul, pct = _stage_c(rowmin, maskf, colmin, cnts, d_dim, k)

    losses = dict(codebook=cb_loss, commitment=cm_loss,
                  unassigned=ul, unassigned_percent=pct)
    return (out_features, assign, losses)


# speedup vs baseline: 1.0406x; 1.0406x over previous
import functools

import jax
import jax.numpy as jnp
from jax.experimental import pallas as pl
from jax.experimental.pallas import tpu as pltpu
from jax.experimental.pallas import tpu_sc as plsc


def _stage_a_body(f2m_ref, maskf_ref, c2_ref, f_ref, cb_ref,
                  assign_ref, rowmin_ref, colmin_ref, rm_s, ra_s,
                  *, bn, bk, kb_total):
    nb = pl.program_id(0)
    kb = pl.program_id(1)
    ns = bk // 128

    fm2 = -2.0 * f_ref[...]

    f2m = f2m_ref[0]
    c2 = c2_ref[0, 0, :]

    @pl.when(kb == 0)
    def _():
        rm_s[...] = jnp.full((bn, 128), 3e38, jnp.float32)
        ra_s[...] = jnp.zeros((bn, 128), jnp.int32)

    m = rm_s[...]
    a = ra_s[...]
    cols = []
    half = bk // 2
    nhs = half // 128
    for h in range(2):
        cbh = cb_ref[pl.ds(h * half, half), :]
        t = jax.lax.dot_general(
            fm2, cbh, (((1,), (1,)), ((), ())),
            preferred_element_type=jnp.float32)
        for sh in range(nhs):
            s = h * nhs + sh
            ts = t[:, sh * 128:(sh + 1) * 128]
            ds = (f2m + ts) + c2[s * 128:(s + 1) * 128][None, :]
            better = ds < m
            m = jnp.where(better, ds, m)
            a = jnp.where(better, jnp.int32(kb * ns + s), a)
            cols.append(jnp.min(ds, axis=0)[None, :])
    rm_s[...] = m
    ra_s[...] = a

    tcol = jnp.concatenate(cols, axis=1)

    @pl.when(nb == 0)
    def _():
        colmin_ref[pl.ds(kb, 1), :] = tcol

    @pl.when(nb > 0)
    def _():
        old = colmin_ref[pl.ds(kb, 1), :]
        colmin_ref[pl.ds(kb, 1), :] = jnp.minimum(old, tcol)

    @pl.when(kb == kb_total - 1)
    def _():
        mfin = rm_s[...]
        afin = ra_s[...]
        tmin = jnp.min(mfin, axis=1)
        lane = jax.lax.broadcasted_iota(jnp.int32, (bn, 128), 1)
        kfull = afin * 128 + lane
        targ = jnp.min(jnp.where(mfin == tmin[:, None], kfull,
                                 jnp.int32(2**30)), axis=1)
        mk = maskf_ref[0, 0, :]
        valid = mk > 0.0
        assign_ref[0, 0, :] = jnp.where(valid, targ, 0)
        rowmin_ref[0, 0, :] = jnp.where(valid, tmin, 0.0)


def _stage_a(features, maskf, cb, f2m, c2, bn, bk):
    n, d_dim = features.shape
    k = cb.shape[0]
    nb_total, kb_total = n // bn, k // bk
    body = functools.partial(_stage_a_body, bn=bn, bk=bk, kb_total=kb_total)
    assign3, rowmin3, colmin2 = pl.pallas_call(
        body,
        grid=(nb_total, kb_total),
        in_specs=[
            pl.BlockSpec((1, bn, 1), lambda nb, kb: (nb, 0, 0)),
            pl.BlockSpec((1, 1, bn), lambda nb, kb: (nb, 0, 0)),
            pl.BlockSpec((1, 1, bk), lambda nb, kb: (kb, 0, 0)),
            pl.BlockSpec((bn, d_dim), lambda nb, kb: (nb, 0)),
            pl.BlockSpec((bk, d_dim), lambda nb, kb: (kb, 0)),
        ],
        out_specs=[
            pl.BlockSpec((1, 1, bn), lambda nb, kb: (nb, 0, 0)),
            pl.BlockSpec((1, 1, bn), lambda nb, kb: (nb, 0, 0)),
            pl.BlockSpec((kb_total, bk), lambda nb, kb: (0, 0)),
        ],
        out_shape=[
            jax.ShapeDtypeStruct((nb_total, 1, bn), jnp.int32),
            jax.ShapeDtypeStruct((nb_total, 1, bn), jnp.float32),
            jax.ShapeDtypeStruct((kb_total, bk), jnp.float32),
        ],
        scratch_shapes=[
            pltpu.VMEM((bn, 128), jnp.float32),
            pltpu.VMEM((bn, 128), jnp.int32),
        ],
    )(f2m.reshape(nb_total, bn, 1), maskf.reshape(nb_total, 1, bn),
      c2.reshape(kb_total, 1, bk), features, cb)
    return (assign3.reshape(n), rowmin3.reshape(n), colmin2.reshape(k))


_SC_CHUNK = 128


def _make_stage_b(n, d_dim, k):
    info = plsc.get_sparse_core_info()
    nc, ns = info.num_cores, info.num_subcores
    rows_per_w = n // (nc * ns)
    chunks = rows_per_w // _SC_CHUNK
    mesh = plsc.VectorSubcoreMesh(core_axis_name="c", subcore_axis_name="s")

    @functools.partial(
        pl.kernel, mesh=mesh,
        out_type=[
            jax.ShapeDtypeStruct((n, d_dim), jnp.float32),
            jax.ShapeDtypeStruct((nc, k), jnp.float32),
        ],
        scratch_types=[
            pltpu.VMEM((rows_per_w,), jnp.int32),
            pltpu.VMEM((rows_per_w,), jnp.float32),
            pltpu.VMEM((2, _SC_CHUNK, d_dim), jnp.float32),
            pltpu.VMEM_SHARED((k,), jnp.float32),
            pltpu.SemaphoreType.DMA,
            pltpu.SemaphoreType.DMA,
            pltpu.SemaphoreType.DMA,
            pltpu.SemaphoreType.DMA,
            pltpu.SemaphoreType.DMA,
    ],
    )
    def stage_b(cb_hbm, assign_hbm, maskf_hbm, zeros_hbm,
                outf_hbm, cnt_hbm, idx_v, val_v, rows_v, shist,
                gs0, gs1, ws0, ws1, hs):
        cid = jax.lax.axis_index("c")
        sid = jax.lax.axis_index("s")
        base = (sid * nc + cid) * rows_per_w
        gsems = (gs0, gs1)
        wsems = (ws0, ws1)

        @pl.when(sid == 0)
        def _():
            pltpu.sync_copy(zeros_hbm, shist)

        pltpu.sync_copy(assign_hbm.at[pl.ds(base, rows_per_w)], idx_v)
        pltpu.sync_copy(maskf_hbm.at[pl.ds(base, rows_per_w)], val_v)
        plsc.subcore_barrier()

        hadd = pltpu.async_copy(val_v, shist.at[idx_v], hs, add=True)

        gets = [None, None]
        puts = [None, None]
        for c in range(chunks):
            b = c % 2
            if puts[b] is not None:
                puts[b].wait()
            gets[b] = pltpu.async_copy(
                cb_hbm.at[idx_v.at[pl.ds(c * _SC_CHUNK, _SC_CHUNK)]],
                rows_v.at[b], gsems[b])
            gets[b].wait()
            puts[b] = pltpu.async_copy(
                rows_v.at[b],
                outf_hbm.at[pl.ds(base + c * _SC_CHUNK, _SC_CHUNK)],
                wsems[b])
        for b in range(2):
            if puts[b] is not None:
                puts[b].wait()
        hadd.wait()

        plsc.subcore_barrier()

        @pl.when(sid == 0)
        def _():
            pltpu.sync_copy(shist, cnt_hbm.at[cid])

    return stage_b


def _stage_c_body(rowmin_ref, maskf_ref, colmin_ref, cnt_ref,
                  cb_loss_ref, cm_loss_ref, ul_ref, pct_ref, *, d_dim, k):
    rm_sum = jnp.sum(rowmin_ref[...])
    nvalid = jnp.sum(maskf_ref[...])
    loss = rm_sum / jnp.float32(d_dim) / jnp.maximum(nvalid, 1.0)
    cb_loss_ref[...] = loss.reshape(1, 1)
    cm_loss_ref[...] = loss.reshape(1, 1)

    cnt = jnp.sum(cnt_ref[...], axis=0)
    colmin = colmin_ref[...].reshape(cnt.shape)
    um = jnp.where(cnt < 1.0, 1.0, 0.0)
    denom = jnp.maximum(jnp.sum(um), 1.0)
    ul = jnp.sum(um * colmin) / jnp.float32(d_dim) / denom
    ul_ref[...] = ul.reshape(1, 1)
    pct = jnp.sum(jnp.where(cnt > 0.0, 1.0, 0.0)) / jnp.float32(k)
    pct_ref[...] = pct.reshape(1, 1)


def _stage_c(rowmin, maskf, colmin, cnts, d_dim, k):
    n = rowmin.shape[0]
    body = functools.partial(_stage_c_body, d_dim=d_dim, k=k)
    outs = pl.pallas_call(
        body,
        out_shape=[jax.ShapeDtypeStruct((1, 1), jnp.float32)] * 4,
    )(rowmin.reshape(n // 128, 128), maskf.reshape(n // 128, 128),
      colmin.reshape(k // 128, 128), cnts)
    return [o.reshape(()) for o in outs]


def kernel(features, mask, codebook, codebook_mean, codebook_scale):
    n, d_dim = features.shape
    k = codebook.shape[0]

    cb = 10.0 * codebook
    scale = jnp.exp(codebook_scale)
    cb = codebook_mean + scale * cb
    f2 = (features ** 2).sum(axis=-1)
    c2 = (cb ** 2).sum(axis=-1)
    maskf = mask.astype(jnp.float32)
    f2m = jnp.where(mask, f2, jnp.float32(1e30))

    assign, rowmin, colmin = _stage_a(features, maskf, cb, f2m, c2,
                                      bn=1024, bk=4096)

    zeros_k = jnp.zeros((k,), jnp.float32)
    out_features, cnts = _make_stage_b(n, d_dim, k)(cb, assign, maskf, zeros_k)

    cb_loss, cm_loss, ul, pct = _stage_c(rowmin, maskf, colmin, cnts, d_dim, k)

    losses = dict(codebook=cb_loss, commitment=cm_loss,
                  unassigned=ul, unassigned_percent=pct)
    return (out_features, assign, losses)


# speedup vs baseline: 1.0850x; 1.0427x over previous
import functools

import jax
import jax.numpy as jnp
from jax.experimental import pallas as pl
from jax.experimental.pallas import tpu as pltpu
from jax.experimental.pallas import tpu_sc as plsc


def _stage_a_body(f2m_ref, maskf_ref, c2_ref, f_ref, cb_ref,
                  assign_ref, rowmin_ref, colmin_ref, rm_s, ra_s,
                  *, bn, bk, kb_total):
    nb = pl.program_id(0)
    kb = pl.program_id(1)
    ns = bk // 128

    fm2 = -2.0 * f_ref[...]

    f2m = f2m_ref[0]
    c2 = c2_ref[0, 0, :]

    @pl.when(kb == 0)
    def _():
        rm_s[...] = jnp.full((bn, 128), 3e38, jnp.float32)
        ra_s[...] = jnp.zeros((bn, 128), jnp.int32)

    m = rm_s[...]
    a = ra_s[...]
    cols = []
    nsplit = max(2, bk // 2048)
    half = bk // nsplit
    nhs = half // 128
    for h in range(nsplit):
        cbh = cb_ref[pl.ds(h * half, half), :]
        t = jax.lax.dot_general(
            fm2, cbh, (((1,), (1,)), ((), ())),
            preferred_element_type=jnp.float32)
        for sh in range(nhs):
            s = h * nhs + sh
            ts = t[:, sh * 128:(sh + 1) * 128]
            ds = (f2m + ts) + c2[s * 128:(s + 1) * 128][None, :]
            better = ds < m
            m = jnp.where(better, ds, m)
            a = jnp.where(better, jnp.int32(kb * ns + s), a)
            cols.append(jnp.min(ds, axis=0)[None, :])
    rm_s[...] = m
    ra_s[...] = a

    tcol = jnp.concatenate(cols, axis=1)

    @pl.when(nb == 0)
    def _():
        colmin_ref[pl.ds(kb, 1), :] = tcol

    @pl.when(nb > 0)
    def _():
        old = colmin_ref[pl.ds(kb, 1), :]
        colmin_ref[pl.ds(kb, 1), :] = jnp.minimum(old, tcol)

    @pl.when(kb == kb_total - 1)
    def _():
        mfin = rm_s[...]
        afin = ra_s[...]
        tmin = jnp.min(mfin, axis=1)
        lane = jax.lax.broadcasted_iota(jnp.int32, (bn, 128), 1)
        kfull = afin * 128 + lane
        targ = jnp.min(jnp.where(mfin == tmin[:, None], kfull,
                                 jnp.int32(2**30)), axis=1)
        mk = maskf_ref[0, 0, :]
        valid = mk > 0.0
        assign_ref[0, 0, :] = jnp.where(valid, targ, 0)
        rowmin_ref[0, 0, :] = jnp.where(valid, tmin, 0.0)


def _stage_a(features, maskf, cb, f2m, c2, bn, bk):
    n, d_dim = features.shape
    k = cb.shape[0]
    nb_total, kb_total = n // bn, k // bk
    body = functools.partial(_stage_a_body, bn=bn, bk=bk, kb_total=kb_total)
    assign3, rowmin3, colmin2 = pl.pallas_call(
        body,
        grid=(nb_total, kb_total),
        in_specs=[
            pl.BlockSpec((1, bn, 1), lambda nb, kb: (nb, 0, 0)),
            pl.BlockSpec((1, 1, bn), lambda nb, kb: (nb, 0, 0)),
            pl.BlockSpec((1, 1, bk), lambda nb, kb: (kb, 0, 0)),
            pl.BlockSpec((bn, d_dim), lambda nb, kb: (nb, 0)),
            pl.BlockSpec((bk, d_dim), lambda nb, kb: (kb, 0)),
        ],
        out_specs=[
            pl.BlockSpec((1, 1, bn), lambda nb, kb: (nb, 0, 0)),
            pl.BlockSpec((1, 1, bn), lambda nb, kb: (nb, 0, 0)),
            pl.BlockSpec((kb_total, bk), lambda nb, kb: (0, 0)),
        ],
        out_shape=[
            jax.ShapeDtypeStruct((nb_total, 1, bn), jnp.int32),
            jax.ShapeDtypeStruct((nb_total, 1, bn), jnp.float32),
            jax.ShapeDtypeStruct((kb_total, bk), jnp.float32),
        ],
        scratch_shapes=[
            pltpu.VMEM((bn, 128), jnp.float32),
            pltpu.VMEM((bn, 128), jnp.int32),
        ],
    )(f2m.reshape(nb_total, bn, 1), maskf.reshape(nb_total, 1, bn),
      c2.reshape(kb_total, 1, bk), features, cb)
    return (assign3.reshape(n), rowmin3.reshape(n), colmin2.reshape(k))


_SC_CHUNK = 128


def _make_stage_b(n, d_dim, k):
    info = plsc.get_sparse_core_info()
    nc, ns = info.num_cores, info.num_subcores
    rows_per_w = n // (nc * ns)
    chunks = rows_per_w // _SC_CHUNK
    mesh = plsc.VectorSubcoreMesh(core_axis_name="c", subcore_axis_name="s")

    @functools.partial(
        pl.kernel, mesh=mesh,
        out_type=[
            jax.ShapeDtypeStruct((n, d_dim), jnp.float32),
            jax.ShapeDtypeStruct((nc, k), jnp.float32),
        ],
        scratch_types=[
            pltpu.VMEM((rows_per_w,), jnp.int32),
            pltpu.VMEM((rows_per_w,), jnp.float32),
            pltpu.VMEM((2, _SC_CHUNK, d_dim), jnp.float32),
            pltpu.VMEM_SHARED((k,), jnp.float32),
            pltpu.SemaphoreType.DMA,
            pltpu.SemaphoreType.DMA,
            pltpu.SemaphoreType.DMA,
            pltpu.SemaphoreType.DMA,
            pltpu.SemaphoreType.DMA,
    ],
    )
    def stage_b(cb_hbm, assign_hbm, maskf_hbm, zeros_hbm,
                outf_hbm, cnt_hbm, idx_v, val_v, rows_v, shist,
                gs0, gs1, ws0, ws1, hs):
        cid = jax.lax.axis_index("c")
        sid = jax.lax.axis_index("s")
        base = (sid * nc + cid) * rows_per_w
        gsems = (gs0, gs1)
        wsems = (ws0, ws1)

        @pl.when(sid == 0)
        def _():
            pltpu.sync_copy(zeros_hbm, shist)

        pltpu.sync_copy(assign_hbm.at[pl.ds(base, rows_per_w)], idx_v)
        pltpu.sync_copy(maskf_hbm.at[pl.ds(base, rows_per_w)], val_v)
        plsc.subcore_barrier()

        hadd = pltpu.async_copy(val_v, shist.at[idx_v], hs, add=True)

        gets = [None, None]
        puts = [None, None]
        for c in range(chunks):
            b = c % 2
            if puts[b] is not None:
                puts[b].wait()
            gets[b] = pltpu.async_copy(
                cb_hbm.at[idx_v.at[pl.ds(c * _SC_CHUNK, _SC_CHUNK)]],
                rows_v.at[b], gsems[b])
            gets[b].wait()
            puts[b] = pltpu.async_copy(
                rows_v.at[b],
                outf_hbm.at[pl.ds(base + c * _SC_CHUNK, _SC_CHUNK)],
                wsems[b])
        for b in range(2):
            if puts[b] is not None:
                puts[b].wait()
        hadd.wait()

        plsc.subcore_barrier()

        @pl.when(sid == 0)
        def _():
            pltpu.sync_copy(shist, cnt_hbm.at[cid])

    return stage_b


def _stage_c_body(rowmin_ref, maskf_ref, colmin_ref, cnt_ref,
                  cb_loss_ref, cm_loss_ref, ul_ref, pct_ref, *, d_dim, k):
    rm_sum = jnp.sum(rowmin_ref[...])
    nvalid = jnp.sum(maskf_ref[...])
    loss = rm_sum / jnp.float32(d_dim) / jnp.maximum(nvalid, 1.0)
    cb_loss_ref[...] = loss.reshape(1, 1)
    cm_loss_ref[...] = loss.reshape(1, 1)

    cnt = jnp.sum(cnt_ref[...], axis=0)
    colmin = colmin_ref[...].reshape(cnt.shape)
    um = jnp.where(cnt < 1.0, 1.0, 0.0)
    denom = jnp.maximum(jnp.sum(um), 1.0)
    ul = jnp.sum(um * colmin) / jnp.float32(d_dim) / denom
    ul_ref[...] = ul.reshape(1, 1)
    pct = jnp.sum(jnp.where(cnt > 0.0, 1.0, 0.0)) / jnp.float32(k)
    pct_ref[...] = pct.reshape(1, 1)


def _stage_c(rowmin, maskf, colmin, cnts, d_dim, k):
    n = rowmin.shape[0]
    body = functools.partial(_stage_c_body, d_dim=d_dim, k=k)
    outs = pl.pallas_call(
        body,
        out_shape=[jax.ShapeDtypeStruct((1, 1), jnp.float32)] * 4,
    )(rowmin.reshape(n // 128, 128), maskf.reshape(n // 128, 128),
      colmin.reshape(k // 128, 128), cnts)
    return [o.reshape(()) for o in outs]


def kernel(features, mask, codebook, codebook_mean, codebook_scale):
    n, d_dim = features.shape
    k = codebook.shape[0]

    cb = 10.0 * codebook
    scale = jnp.exp(codebook_scale)
    cb = codebook_mean + scale * cb
    f2 = (features ** 2).sum(axis=-1)
    c2 = (cb ** 2).sum(axis=-1)
    maskf = mask.astype(jnp.float32)
    f2m = jnp.where(mask, f2, jnp.float32(1e30))

    assign, rowmin, colmin = _stage_a(features, maskf, cb, f2m, c2,
                                      bn=1024, bk=8192)

    zeros_k = jnp.zeros((k,), jnp.float32)
    out_features, cnts = _make_stage_b(n, d_dim, k)(cb, assign, maskf, zeros_k)

    cb_loss, cm_loss, ul, pct = _stage_c(rowmin, maskf, colmin, cnts, d_dim, k)

    losses = dict(codebook=cb_loss, commitment=cm_loss,
                  unassigned=ul, unassigned_percent=pct)
    return (out_features, assign, losses)


# speedup vs baseline: 1.1599x; 1.0690x over previous
import functools

import jax
import jax.numpy as jnp
from jax.experimental import pallas as pl
from jax.experimental.pallas import tpu as pltpu
from jax.experimental.pallas import tpu_sc as plsc


def _stage_a_body(f2m_ref, maskf_ref, c2_ref, f_ref, cb_ref,
                  assign_ref, rowmin_ref, colmin_ref, rm_s, ra_s,
                  *, bn, bk, kb_total):
    nb = pl.program_id(0)
    kb = pl.program_id(1)
    ns = bk // 128

    fm2 = -2.0 * f_ref[...]

    f2m = f2m_ref[0]
    c2 = c2_ref[0, 0, :]

    @pl.when(kb == 0)
    def _():
        rm_s[...] = jnp.full((bn, 128), 3e38, jnp.float32)
        ra_s[...] = jnp.zeros((bn, 128), jnp.int32)

    m = rm_s[...]
    a = ra_s[...]
    cols = []
    nsplit = max(2, (bn * bk * 4) // (8 * 1024 * 1024))
    half = bk // nsplit
    nhs = half // 128
    for h in range(nsplit):
        cbh = cb_ref[pl.ds(h * half, half), :]
        t = jax.lax.dot_general(
            fm2, cbh, (((1,), (1,)), ((), ())),
            preferred_element_type=jnp.float32)
        for sh in range(nhs):
            s = h * nhs + sh
            ts = t[:, sh * 128:(sh + 1) * 128]
            ds = (f2m + ts) + c2[s * 128:(s + 1) * 128][None, :]
            better = ds < m
            m = jnp.where(better, ds, m)
            a = jnp.where(better, jnp.int32(kb * ns + s), a)
            cols.append(jnp.min(ds, axis=0)[None, :])
    rm_s[...] = m
    ra_s[...] = a

    tcol = jnp.concatenate(cols, axis=1)

    @pl.when(nb == 0)
    def _():
        colmin_ref[pl.ds(kb, 1), :] = tcol

    @pl.when(nb > 0)
    def _():
        old = colmin_ref[pl.ds(kb, 1), :]
        colmin_ref[pl.ds(kb, 1), :] = jnp.minimum(old, tcol)

    @pl.when(kb == kb_total - 1)
    def _():
        mfin = rm_s[...]
        afin = ra_s[...]
        tmin = jnp.min(mfin, axis=1)
        lane = jax.lax.broadcasted_iota(jnp.int32, (bn, 128), 1)
        kfull = afin * 128 + lane
        targ = jnp.min(jnp.where(mfin == tmin[:, None], kfull,
                                 jnp.int32(2**30)), axis=1)
        mk = maskf_ref[0, 0, :]
        valid = mk > 0.0
        assign_ref[0, 0, :] = jnp.where(valid, targ, 0)
        rowmin_ref[0, 0, :] = jnp.where(valid, tmin, 0.0)


def _stage_a(features, maskf, cb, f2m, c2, bn, bk):
    n, d_dim = features.shape
    k = cb.shape[0]
    nb_total, kb_total = n // bn, k // bk
    body = functools.partial(_stage_a_body, bn=bn, bk=bk, kb_total=kb_total)
    assign3, rowmin3, colmin2 = pl.pallas_call(
        body,
        grid=(nb_total, kb_total),
        in_specs=[
            pl.BlockSpec((1, bn, 1), lambda nb, kb: (nb, 0, 0)),
            pl.BlockSpec((1, 1, bn), lambda nb, kb: (nb, 0, 0)),
            pl.BlockSpec((1, 1, bk), lambda nb, kb: (kb, 0, 0)),
            pl.BlockSpec((bn, d_dim), lambda nb, kb: (nb, 0)),
            pl.BlockSpec((bk, d_dim), lambda nb, kb: (kb, 0)),
        ],
        out_specs=[
            pl.BlockSpec((1, 1, bn), lambda nb, kb: (nb, 0, 0)),
            pl.BlockSpec((1, 1, bn), lambda nb, kb: (nb, 0, 0)),
            pl.BlockSpec((kb_total, bk), lambda nb, kb: (0, 0)),
        ],
        out_shape=[
            jax.ShapeDtypeStruct((nb_total, 1, bn), jnp.int32),
            jax.ShapeDtypeStruct((nb_total, 1, bn), jnp.float32),
            jax.ShapeDtypeStruct((kb_total, bk), jnp.float32),
        ],
        scratch_shapes=[
            pltpu.VMEM((bn, 128), jnp.float32),
            pltpu.VMEM((bn, 128), jnp.int32),
        ],
    )(f2m.reshape(nb_total, bn, 1), maskf.reshape(nb_total, 1, bn),
      c2.reshape(kb_total, 1, bk), features, cb)
    return (assign3.reshape(n), rowmin3.reshape(n), colmin2.reshape(k))


_SC_CHUNK = 128


def _make_stage_b(n, d_dim, k):
    info = plsc.get_sparse_core_info()
    nc, ns = info.num_cores, info.num_subcores
    rows_per_w = n // (nc * ns)
    chunks = rows_per_w // _SC_CHUNK
    mesh = plsc.VectorSubcoreMesh(core_axis_name="c", subcore_axis_name="s")

    @functools.partial(
        pl.kernel, mesh=mesh,
        out_type=[
            jax.ShapeDtypeStruct((n, d_dim), jnp.float32),
            jax.ShapeDtypeStruct((nc, k), jnp.float32),
        ],
        scratch_types=[
            pltpu.VMEM((rows_per_w,), jnp.int32),
            pltpu.VMEM((rows_per_w,), jnp.float32),
            pltpu.VMEM((2, _SC_CHUNK, d_dim), jnp.float32),
            pltpu.VMEM_SHARED((k,), jnp.float32),
            pltpu.SemaphoreType.DMA,
            pltpu.SemaphoreType.DMA,
            pltpu.SemaphoreType.DMA,
            pltpu.SemaphoreType.DMA,
            pltpu.SemaphoreType.DMA,
    ],
    )
    def stage_b(cb_hbm, assign_hbm, maskf_hbm, zeros_hbm,
                outf_hbm, cnt_hbm, idx_v, val_v, rows_v, shist,
                gs0, gs1, ws0, ws1, hs):
        cid = jax.lax.axis_index("c")
        sid = jax.lax.axis_index("s")
        base = (sid * nc + cid) * rows_per_w
        gsems = (gs0, gs1)
        wsems = (ws0, ws1)

        @pl.when(sid == 0)
        def _():
            pltpu.sync_copy(zeros_hbm, shist)

        pltpu.sync_copy(assign_hbm.at[pl.ds(base, rows_per_w)], idx_v)
        pltpu.sync_copy(maskf_hbm.at[pl.ds(base, rows_per_w)], val_v)
        plsc.subcore_barrier()

        hadd = pltpu.async_copy(val_v, shist.at[idx_v], hs, add=True)

        gets = [None, None]
        puts = [None, None]
        for c in range(chunks):
            b = c % 2
            if puts[b] is not None:
                puts[b].wait()
            gets[b] = pltpu.async_copy(
                cb_hbm.at[idx_v.at[pl.ds(c * _SC_CHUNK, _SC_CHUNK)]],
                rows_v.at[b], gsems[b])
            gets[b].wait()
            puts[b] = pltpu.async_copy(
                rows_v.at[b],
                outf_hbm.at[pl.ds(base + c * _SC_CHUNK, _SC_CHUNK)],
                wsems[b])
        for b in range(2):
            if puts[b] is not None:
                puts[b].wait()
        hadd.wait()

        plsc.subcore_barrier()

        @pl.when(sid == 0)
        def _():
            pltpu.sync_copy(shist, cnt_hbm.at[cid])

    return stage_b


def _stage_c_body(rowmin_ref, maskf_ref, colmin_ref, cnt_ref,
                  cb_loss_ref, cm_loss_ref, ul_ref, pct_ref, *, d_dim, k):
    rm_sum = jnp.sum(rowmin_ref[...])
    nvalid = jnp.sum(maskf_ref[...])
    loss = rm_sum / jnp.float32(d_dim) / jnp.maximum(nvalid, 1.0)
    cb_loss_ref[...] = loss.reshape(1, 1)
    cm_loss_ref[...] = loss.reshape(1, 1)

    cnt = jnp.sum(cnt_ref[...], axis=0)
    colmin = colmin_ref[...].reshape(cnt.shape)
    um = jnp.where(cnt < 1.0, 1.0, 0.0)
    denom = jnp.maximum(jnp.sum(um), 1.0)
    ul = jnp.sum(um * colmin) / jnp.float32(d_dim) / denom
    ul_ref[...] = ul.reshape(1, 1)
    pct = jnp.sum(jnp.where(cnt > 0.0, 1.0, 0.0)) / jnp.float32(k)
    pct_ref[...] = pct.reshape(1, 1)


def _stage_c(rowmin, maskf, colmin, cnts, d_dim, k):
    n = rowmin.shape[0]
    body = functools.partial(_stage_c_body, d_dim=d_dim, k=k)
    outs = pl.pallas_call(
        body,
        out_shape=[jax.ShapeDtypeStruct((1, 1), jnp.float32)] * 4,
    )(rowmin.reshape(n // 128, 128), maskf.reshape(n // 128, 128),
      colmin.reshape(k // 128, 128), cnts)
    return [o.reshape(()) for o in outs]


def kernel(features, mask, codebook, codebook_mean, codebook_scale):
    n, d_dim = features.shape
    k = codebook.shape[0]

    cb = 10.0 * codebook
    scale = jnp.exp(codebook_scale)
    cb = codebook_mean + scale * cb
    f2 = (features ** 2).sum(axis=-1)
    c2 = (cb ** 2).sum(axis=-1)
    maskf = mask.astype(jnp.float32)
    f2m = jnp.where(mask, f2, jnp.float32(1e30))

    assign, rowmin, colmin = _stage_a(features, maskf, cb, f2m, c2,
                                      bn=2048, bk=8192)

    zeros_k = jnp.zeros((k,), jnp.float32)
    out_features, cnts = _make_stage_b(n, d_dim, k)(cb, assign, maskf, zeros_k)

    cb_loss, cm_loss, ul, pct = _stage_c(rowmin, maskf, colmin, cnts, d_dim, k)

    losses = dict(codebook=cb_loss, commitment=cm_loss,
                  unassigned=ul, unassigned_percent=pct)
    return (out_features, assign, losses)
